# baseline (speedup 1.0000x reference)
import functools

import jax
import jax.numpy as jnp
from jax import lax
from jax.experimental import pallas as pl
from jax.experimental.pallas import tpu as pltpu
from jax.experimental.pallas import tpu_sc as plsc

BATCH = 16384
DIM = 64
NC = 2
NS = 16
NW = NC * NS
BPW = BATCH // NW
LANES = 16
NGROUPS = BPW // LANES

NE = 1000000
WIN = 512
NWIN = 1954
LAST_WIN = NWIN - 1
NSLOT = 2 * BATCH
ROWS_PAD = NSLOT + 8
PK_PAD = 31744 * 65536
STRIP = 2048
SEG = 8192
STAG_ROWS = 128

_mesh = plsc.VectorSubcoreMesh(core_axis_name="c", subcore_axis_name="s")


def _sqrt16(x):
    i = plsc.bitcast(x, jnp.int32)
    i = jnp.int32(0x5F3759DF) - lax.shift_right_arithmetic(i, 1)
    y = plsc.bitcast(i, jnp.float32)
    for _ in range(3):
        y = y * (jnp.float32(1.5) - jnp.float32(0.5) * x * y * y)
    return x * y


@functools.partial(
    pl.kernel,
    out_type=jax.ShapeDtypeStruct((NSLOT,), jnp.int32),
    mesh=_mesh,
    compiler_params=pltpu.CompilerParams(
        needs_layout_passes=False, use_tc_tiling_on_sc=False),
    scratch_types=[
        pltpu.VMEM((BPW,), jnp.int32),
        pltpu.VMEM((BPW,), jnp.int32),
        pltpu.VMEM((BPW,), jnp.int32),
        pltpu.VMEM((BPW,), jnp.int32),
        pltpu.SemaphoreType.DMA,
        pltpu.SemaphoreType.DMA,
    ],
)
def _ids_kernel(xs_hbm, ys_hbm, gid_hbm, ids_hbm, xv, yv, xe, ye, sem1, sem2):
    wid = lax.axis_index("s") * NC + lax.axis_index("c")
    base = wid * BPW
    pltpu.sync_copy(xs_hbm.at[pl.ds(base, BPW)], xv)
    pltpu.sync_copy(ys_hbm.at[pl.ds(base, BPW)], yv)
    cx = pltpu.async_copy(gid_hbm.at[xv], xe, sem1)
    cy = pltpu.async_copy(gid_hbm.at[yv], ye, sem2)
    cx.wait()
    cy.wait()
    pltpu.sync_copy(xe, ids_hbm.at[pl.ds(base, BPW)])
    pltpu.sync_copy(ye, ids_hbm.at[pl.ds(BATCH + base, BPW)])


@functools.partial(
    pl.kernel,
    out_type=jax.ShapeDtypeStruct((ROWS_PAD, 128), jnp.float32),
    mesh=_mesh,
    compiler_params=pltpu.CompilerParams(
        needs_layout_passes=False, use_tc_tiling_on_sc=True),
    scratch_types=[
        pltpu.VMEM((DIM, WIN), jnp.float32),
        pltpu.VMEM((DIM, WIN), jnp.float32),
        pltpu.VMEM((NSLOT + LANES,), jnp.int32),
        pltpu.VMEM((SEG + LANES,), jnp.int32),
        pltpu.VMEM((STRIP,), jnp.int32),
        pltpu.VMEM((STAG_ROWS, 128), jnp.float32),
        pltpu.VMEM((STAG_ROWS,), jnp.int32),
        pltpu.SMEM((1,), jnp.int32),
        pltpu.SemaphoreType.DMA,
        pltpu.SemaphoreType.DMA,
        pltpu.SemaphoreType.DMA,
    ],
)
def _rows_kernel(ids_hbm, embt_hbm, tail_hbm, rows_hbm, chunk0, chunk1, sel,
                 clist, strip, stag, slotbuf, nstag, semw0, semw1, sems):
    wid = lax.axis_index("s") * NC + lax.axis_index("c")
    wstart = wid * 61 + jnp.minimum(wid, 2)
    wcount = 61 + jnp.where(wid < 2, 1, 0)
    e0 = wstart * WIN
    lane_iota = lax.iota(jnp.int32, LANES)
    nstag[0] = jnp.int32(0)

    def strip_body(s, cnt):
        pltpu.sync_copy(ids_hbm.at[pl.ds(s * STRIP, STRIP)], strip)

        def vreg_body(k, cnt):
            e = strip[pl.ds(k * LANES, LANES)]
            le = e - e0
            m = (le >= 0) & (le < wcount * WIN)
            slot = s * STRIP + k * LANES + lane_iota
            pk = lax.shift_left(le, 16) + slot
            plsc.store_compressed(sel.at[pl.ds(cnt, LANES)], pk, mask=m)
            return cnt + plsc.all_reduce_population_count(m)[0]

        return lax.fori_loop(0, STRIP // LANES, vreg_body, cnt)

    nsel = lax.fori_loop(0, NSLOT // STRIP, strip_body, jnp.int32(0))
    sel[pl.ds(nsel, LANES)] = jnp.full((LANES,), PK_PAD, jnp.int32)
    nselv = pl.cdiv(nsel, LANES)

    def win_copy(cl, buf, semw):
        gw = wstart + cl

        def fire_or_wait(start):
            @pl.when(gw != LAST_WIN)
            def _():
                d = pltpu.make_async_copy(
                    embt_hbm.at[:, pl.ds(gw * WIN, WIN)], buf, semw)
                d.start() if start else d.wait()

            @pl.when(gw == LAST_WIN)
            def _():
                d = pltpu.make_async_copy(
                    tail_hbm, buf.at[:, pl.ds(0, 128)], semw)
                d.start() if start else d.wait()

        return fire_or_wait

    def flush():
        pltpu.async_copy(stag, rows_hbm.at[slotbuf], sems).wait()
        nstag[0] = jnp.int32(0)

    def process(cl, chunk):
        def seg_body(sg, carry):
            segbase = sg * (SEG // LANES)
            nv = jnp.minimum(nselv - segbase, SEG // LANES)

            def match_body(k, mcnt):
                pk = sel[pl.ds((segbase + k) * LANES, LANES)]
                m = lax.shift_right_arithmetic(pk, 25) == cl
                plsc.store_compressed(clist.at[pl.ds(mcnt, LANES)], pk,
                                      mask=m)
                return mcnt + plsc.all_reduce_population_count(m)[0]

            mcnt = lax.fori_loop(0, nv, match_body, jnp.int32(0))
            clist[pl.ds(mcnt, LANES)] = jnp.full((LANES,), NSLOT, jnp.int32)

            def extract_body(k, carry2):
                pk = clist[pl.ds(k * LANES, LANES)]
                slot = pk & jnp.int32(0xFFFF)
                col = lax.shift_right_arithmetic(pk, 16) & jnp.int32(WIN - 1)
                ns = nstag[0]
                rowidx = ns + lane_iota
                for d in range(DIM):
                    vals = plsc.load_gather(
                        chunk, [jnp.full((LANES,), d, jnp.int32), col])
                    plsc.store_scatter(
                        stag, [rowidx, jnp.full((LANES,), d, jnp.int32)],
                        vals)
                slotbuf[pl.ds(ns, LANES)] = slot
                nstag[0] = ns + LANES

                @pl.when(ns + LANES == STAG_ROWS)
                def _():
                    flush()

                return carry2

            lax.fori_loop(0, pl.cdiv(mcnt, LANES), extract_body, 0)
            return carry

        lax.fori_loop(0, pl.cdiv(nsel, SEG), seg_body, 0)

    win_copy(0, chunk0, semw0)(True)

    def outer(o, carry):
        for b, buf, semw, obuf, osemw in ((0, chunk0, semw0, chunk1, semw1),
                                          (1, chunk1, semw1, chunk0, semw0)):
            cl = 2 * o + b

            @pl.when(cl < wcount)
            def _():
                @pl.when(cl + 1 < wcount)
                def _():
                    win_copy(cl + 1, obuf, osemw)(True)

                win_copy(cl, buf, semw)(False)
                process(cl, buf)

        return carry

    lax.fori_loop(0, 31, outer, 0)

    ns_end = nstag[0]
    for j in range(STAG_ROWS // LANES):
        @pl.when(j * LANES >= ns_end)
        def _():
            slotbuf[pl.ds(j * LANES, LANES)] = jnp.full(
                (LANES,), NSLOT, jnp.int32)

    flush()


@functools.partial(
    pl.kernel,
    out_type=jax.ShapeDtypeStruct((BATCH,), jnp.float32),
    mesh=_mesh,
    compiler_params=pltpu.CompilerParams(
        needs_layout_passes=False, use_tc_tiling_on_sc=True),
    scratch_types=[
        pltpu.VMEM((128, 128), jnp.float32),
        pltpu.VMEM((128, 128), jnp.float32),
        pltpu.VMEM((DIM,), jnp.float32),
        pltpu.VMEM((BPW,), jnp.float32),
        pltpu.SemaphoreType.DMA,
        pltpu.SemaphoreType.DMA,
    ],
)
def _score_kernel(rows_hbm, rel_hbm, out_hbm, hv, tv, rv, ov, sem1, sem2):
    wid = lax.axis_index("s") * NC + lax.axis_index("c")
    base = wid * BPW
    pltpu.sync_copy(rel_hbm, rv)
    rchunks = [rv[pl.ds(j * LANES, LANES)] for j in range(DIM // LANES)]
    lane_iota = lax.iota(jnp.int32, LANES)
    perms = {d: lane_iota ^ d for d in (1, 2, 4, 8)}
    masks = {d: (lane_iota & d) == 0 for d in (1, 2, 4, 8)}

    def combine(a, b, dist):
        m = masks[dist]
        w = jnp.where(m, b, a)
        _, wp = plsc.sort_key_val(perms[dist], w)
        return jnp.where(m, a, b) + wp

    def sub_body(j, carry):
        s0 = base + j * 128
        ch = pltpu.async_copy(rows_hbm.at[pl.ds(s0, 128), :], hv, sem1)
        ct = pltpu.async_copy(rows_hbm.at[pl.ds(BATCH + s0, 128), :], tv, sem2)
        ch.wait()
        ct.wait()

        def group_body(g, carry2):
            svecs = []
            for p in range(LANES):
                i = g * LANES + p
                s = None
                for q in range(DIM // LANES):
                    hq = hv[i, pl.ds(q * LANES, LANES)]
                    tq = tv[i, pl.ds(q * LANES, LANES)]
                    dd = hq - tq + rchunks[q]
                    s = dd * dd if s is None else s + dd * dd
                svecs.append(s)
            dist = 1
            while len(svecs) > 1:
                svecs = [combine(svecs[k], svecs[k + 1], dist)
                         for k in range(0, len(svecs), 2)]
                dist *= 2
            ov[pl.ds(j * 128 + g * LANES, LANES)] = _sqrt16(svecs[0])
            return carry2

        lax.fori_loop(0, 128 // LANES, group_body, 0)
        return carry

    lax.fori_loop(0, BPW // 128, sub_body, 0)
    pltpu.sync_copy(ov, out_hbm.at[pl.ds(base, BPW)])


def kernel(data, graph_ids, entity_emb, relation_emb):
    xs = data[:, 0]
    ys = data[:, 1]
    embt = entity_emb.T
    tail = jnp.pad(entity_emb[LAST_WIN * WIN:].T, ((0, 0), (0, 64)))
    rel = relation_emb.reshape(DIM)
    ids = _ids_kernel(xs, ys, graph_ids)
    rows = _rows_kernel(ids, embt, tail)
    scores = _score_kernel(rows, rel)
    return scores.reshape(BATCH, 1)


# speedup vs baseline: 1.6778x; 1.6778x over previous
import functools

import jax
import jax.numpy as jnp
from jax import lax
from jax.experimental import pallas as pl
from jax.experimental.pallas import tpu as pltpu
from jax.experimental.pallas import tpu_sc as plsc

BATCH = 16384
DIM = 64
NC = 2
NS = 16
NW = NC * NS
BPW = BATCH // NW
LANES = 16
NGROUPS = BPW // LANES

_mesh = plsc.VectorSubcoreMesh(core_axis_name="c", subcore_axis_name="s")


def _sqrt16(x):
    i = plsc.bitcast(x, jnp.int32)
    i = jnp.int32(0x5F3759DF) - lax.shift_right_arithmetic(i, 1)
    y = plsc.bitcast(i, jnp.float32)
    for _ in range(3):
        y = y * (jnp.float32(1.5) - jnp.float32(0.5) * x * y * y)
    return x * y


@functools.partial(
    pl.kernel,
    out_type=jax.ShapeDtypeStruct((BATCH,), jnp.float32),
    mesh=_mesh,
    compiler_params=pltpu.CompilerParams(
        needs_layout_passes=False, use_tc_tiling_on_sc=False),
    scratch_types=[
        pltpu.VMEM((BPW,), jnp.int32),
        pltpu.VMEM((BPW,), jnp.int32),
        pltpu.VMEM((BPW,), jnp.int32),
        pltpu.VMEM((BPW,), jnp.int32),
        pltpu.VMEM((BPW, DIM), jnp.float32),
        pltpu.VMEM((BPW, DIM), jnp.float32),
        pltpu.VMEM((DIM,), jnp.float32),
        pltpu.VMEM((BPW,), jnp.float32),
        pltpu.SemaphoreType.DMA,
        pltpu.SemaphoreType.DMA,
    ],
)
def _score_kernel(xs_hbm, ys_hbm, gid_hbm, emb_hbm, rel_hbm, out_hbm,
                  xv, yv, xe, ye, hv, tv, rv, ov, sem1, sem2):
    wid = lax.axis_index("s") * NC + lax.axis_index("c")
    base = wid * BPW

    pltpu.sync_copy(xs_hbm.at[pl.ds(base, BPW)], xv)
    pltpu.sync_copy(ys_hbm.at[pl.ds(base, BPW)], yv)
    pltpu.sync_copy(rel_hbm, rv)

    cx = pltpu.async_copy(gid_hbm.at[xv], xe, sem1)
    cy = pltpu.async_copy(gid_hbm.at[yv], ye, sem2)
    cx.wait()
    cy.wait()

    ch = pltpu.async_copy(emb_hbm.at[xe], hv, sem1)
    ct = pltpu.async_copy(emb_hbm.at[ye], tv, sem2)
    ch.wait()
    ct.wait()

    rchunks = [rv[pl.ds(c * LANES, LANES)] for c in range(DIM // LANES)]
    lane_iota = lax.iota(jnp.int32, LANES)
    perms = {d: lane_iota ^ d for d in (1, 2, 4, 8)}
    masks = {d: (lane_iota & d) == 0 for d in (1, 2, 4, 8)}

    def combine(a, b, dist):
        m = masks[dist]
        w = jnp.where(m, b, a)
        _, wp = plsc.sort_key_val(perms[dist], w)
        return jnp.where(m, a, b) + wp

    def group_body(g, carry):
        svecs = []
        for p in range(LANES):
            i = g * LANES + p
            s = None
            for j in range(DIM // LANES):
                hj = hv[i, pl.ds(j * LANES, LANES)]
                tj = tv[i, pl.ds(j * LANES, LANES)]
                d = hj - tj + rchunks[j]
                s = d * d if s is None else s + d * d
            svecs.append(s)
        dist = 1
        while len(svecs) > 1:
            svecs = [combine(svecs[k], svecs[k + 1], dist)
                     for k in range(0, len(svecs), 2)]
            dist *= 2
        ov[pl.ds(g * LANES, LANES)] = _sqrt16(svecs[0])
        return carry

    lax.fori_loop(0, NGROUPS, group_body, 0)

    pltpu.sync_copy(ov, out_hbm.at[pl.ds(base, BPW)])


def kernel(data, graph_ids, entity_emb, relation_emb):
    xs = data[:, 0]
    ys = data[:, 1]
    rel = relation_emb.reshape(DIM)
    scores = _score_kernel(xs, ys, graph_ids, entity_emb, rel)
    return scores.reshape(BATCH, 1)
